# Initial kernel scaffold; baseline (speedup 1.0000x reference)
import jax
import jax.numpy as jnp
from jax.experimental import pallas as pl


def kernel(x, W_in, W_l, b_l, W_r, W_out, b_out):
    raise NotImplementedError("write your pallas kernel here")


# speedup vs baseline: 40.2366x; 40.2366x over previous
import functools

import jax
import jax.numpy as jnp
from jax import lax
from jax.experimental import pallas as pl
from jax.experimental.pallas import tpu as pltpu

N = 10000
D = 128
H = 128
NP = 10240
T = 1024
G = NP // T
K_RANK = 320000
NBINS = 32

_INTERPRET = False


def _prep_kernel(x_ref, w_ref, xn_ref, h_ref):
    x = x_ref[...]
    nrm = jnp.sqrt(jnp.sum(x * x, axis=1, keepdims=True))
    xn_ref[...] = x / jnp.maximum(nrm, 1e-8)
    h_ref[...] = jnp.maximum(
        jnp.dot(x, w_ref[...], preferred_element_type=jnp.float32), 0.0)


def _sim_hist_kernel(xn_i_ref, xn_j_ref, sim_ref, hist_ref, hacc_ref):
    i = pl.program_id(0)
    j = pl.program_id(1)
    s = jnp.dot(xn_i_ref[...], xn_j_ref[...].T,
                preferred_element_type=jnp.float32)
    s = jnp.abs(s)
    row = i * T + lax.broadcasted_iota(jnp.int32, (T, T), 0)
    col = j * T + lax.broadcasted_iota(jnp.int32, (T, T), 1)
    s = jnp.where(row == col, 0.0, s)
    sim_ref[...] = s

    bits = lax.bitcast_convert_type(s, jnp.int32)
    first = jnp.logical_and(i == 0, j == 0)
    for b in range(NBINS):
        t = b << 25
        cnt = jnp.sum((bits >= t).astype(jnp.int32), axis=0)
        prev = jnp.where(first, 0, hacc_ref[b, :])
        hacc_ref[b, :] = prev + cnt

    @pl.when(jnp.logical_and(i == G - 1, j == G - 1))
    def _():
        for b in range(NBINS):
            hist_ref[0, b] = jnp.sum(hacc_ref[b, :])


def _hist_kernel(lo_ref, sim_ref, hist_ref, hacc_ref, *, shift):
    i = pl.program_id(0)
    j = pl.program_id(1)
    lo = lo_ref[0, 0]
    bits = lax.bitcast_convert_type(sim_ref[...], jnp.int32)
    first = jnp.logical_and(i == 0, j == 0)
    for b in range(NBINS):
        t = lo + (b << shift)
        cnt = jnp.sum((bits >= t).astype(jnp.int32), axis=0)
        prev = jnp.where(first, 0, hacc_ref[b, :])
        hacc_ref[b, :] = prev + cnt

    @pl.when(jnp.logical_and(i == G - 1, j == G - 1))
    def _():
        for b in range(NBINS):
            hist_ref[0, b] = jnp.sum(hacc_ref[b, :])


def _agg_kernel(eps_ref, bout_ref, sim_ref, h_ref, wl_ref, bl_ref, wr_ref,
                wout_ref, hk_ref, out_ref, acc_ref, cnt_ref):
    i = pl.program_id(1)
    eps = eps_ref[0, 0]
    col = i * T + lax.broadcasted_iota(jnp.int32, (T, T), 1)
    maskf = jnp.where(jnp.logical_and(sim_ref[...] >= eps, col < N), 1.0, 0.0)
    part = jnp.dot(maskf, h_ref[...], preferred_element_type=jnp.float32)
    pcnt = jnp.sum(maskf, axis=1, keepdims=True)

    @pl.when(i == 0)
    def _():
        acc_ref[...] = part
        cnt_ref[...] = pcnt

    @pl.when(i > 0)
    def _():
        acc_ref[...] += part
        cnt_ref[...] += pcnt

    @pl.when(i == G - 1)
    def _():
        agg = acc_ref[...] / jnp.maximum(cnt_ref[...], 1.0)
        z = (jnp.dot(agg, wl_ref[...], preferred_element_type=jnp.float32)
             + bl_ref[...]
             + jnp.dot(hk_ref[...], wr_ref[...],
                       preferred_element_type=jnp.float32))
        h2 = jnp.maximum(z, 0.0)
        out_ref[...] = (jnp.dot(h2, wout_ref[...],
                                preferred_element_type=jnp.float32)
                        + bout_ref[0, 0])


def _select(c, lo, shift):
    jstar = jnp.sum((c[0] >= K_RANK).astype(jnp.int32)) - 1
    return lo + (jstar << shift)


def kernel(x, W_in, W_l, b_l, W_r, W_out, b_out):
    f32 = jnp.float32
    xp = jnp.zeros((NP, D), f32).at[:N].set(x.astype(f32))

    xn, h = pl.pallas_call(
        _prep_kernel,
        out_shape=(jax.ShapeDtypeStruct((NP, D), f32),
                   jax.ShapeDtypeStruct((NP, D), f32)),
        interpret=_INTERPRET,
    )(xp, W_in.astype(f32))

    sim, c1 = pl.pallas_call(
        _sim_hist_kernel,
        grid=(G, G),
        in_specs=[pl.BlockSpec((T, D), lambda i, j: (i, 0)),
                  pl.BlockSpec((T, D), lambda i, j: (j, 0))],
        out_specs=(pl.BlockSpec((T, T), lambda i, j: (i, j)),
                   pl.BlockSpec((1, NBINS), lambda i, j: (0, 0),
                                memory_space=pltpu.SMEM)),
        out_shape=(jax.ShapeDtypeStruct((NP, NP), f32),
                   jax.ShapeDtypeStruct((1, NBINS), jnp.int32)),
        scratch_shapes=[pltpu.VMEM((NBINS, T), jnp.int32)],
        compiler_params=pltpu.CompilerParams(
            dimension_semantics=("arbitrary", "arbitrary")),
        interpret=_INTERPRET,
    )(xn, xn)

    lo = _select(c1, jnp.int32(0), 25)
    for shift in (20, 15, 10, 5, 0):
        c = pl.pallas_call(
            functools.partial(_hist_kernel, shift=shift),
            grid=(G, G),
            in_specs=[pl.BlockSpec((1, 1), lambda i, j: (0, 0),
                                   memory_space=pltpu.SMEM),
                      pl.BlockSpec((T, T), lambda i, j: (i, j))],
            out_specs=pl.BlockSpec((1, NBINS), lambda i, j: (0, 0),
                                   memory_space=pltpu.SMEM),
            out_shape=jax.ShapeDtypeStruct((1, NBINS), jnp.int32),
            scratch_shapes=[pltpu.VMEM((NBINS, T), jnp.int32)],
            compiler_params=pltpu.CompilerParams(
                dimension_semantics=("arbitrary", "arbitrary")),
            interpret=_INTERPRET,
        )(lo.reshape(1, 1), sim)
        lo = _select(c, lo, shift)

    eps = lax.bitcast_convert_type(lo, f32)

    out = pl.pallas_call(
        _agg_kernel,
        grid=(G, G),
        in_specs=[pl.BlockSpec((1, 1), lambda k, i: (0, 0),
                               memory_space=pltpu.SMEM),
                  pl.BlockSpec((1, 1), lambda k, i: (0, 0),
                               memory_space=pltpu.SMEM),
                  pl.BlockSpec((T, T), lambda k, i: (k, i)),
                  pl.BlockSpec((T, H), lambda k, i: (i, 0)),
                  pl.BlockSpec((H, H), lambda k, i: (0, 0)),
                  pl.BlockSpec((1, H), lambda k, i: (0, 0)),
                  pl.BlockSpec((H, H), lambda k, i: (0, 0)),
                  pl.BlockSpec((H, 1), lambda k, i: (0, 0)),
                  pl.BlockSpec((T, H), lambda k, i: (k, 0))],
        out_specs=pl.BlockSpec((T, 1), lambda k, i: (k, 0)),
        out_shape=jax.ShapeDtypeStruct((NP, 1), f32),
        scratch_shapes=[pltpu.VMEM((T, H), f32),
                        pltpu.VMEM((T, 1), f32)],
        compiler_params=pltpu.CompilerParams(
            dimension_semantics=("arbitrary", "arbitrary")),
        interpret=_INTERPRET,
    )(eps.reshape(1, 1), b_out.astype(f32).reshape(1, 1), sim, h,
      W_l.astype(f32), b_l.astype(f32).reshape(1, H), W_r.astype(f32),
      W_out.astype(f32), h)

    return out[:N]


# speedup vs baseline: 76.6699x; 1.9055x over previous
import functools

import jax
import jax.numpy as jnp
from jax import lax
from jax.experimental import pallas as pl
from jax.experimental.pallas import tpu as pltpu

N = 10000
D = 128
H = 128
NP = 10240
T = 1024
G = NP // T
K_RANK = 320000
NBINS = 8
SHIFTS = (27, 24, 21, 18, 15, 12, 9, 6, 3, 0)

_INTERPRET = False


def _prep_kernel(x_ref, w_ref, xn_ref, h_ref):
    x = x_ref[...]
    nrm = jnp.sqrt(jnp.sum(x * x, axis=1, keepdims=True))
    xn_ref[...] = x / jnp.maximum(nrm, 1e-8)
    h_ref[...] = jnp.maximum(
        jnp.dot(x, w_ref[...], preferred_element_type=jnp.float32), 0.0)


def _sim_tile(xn_i, xn_j, i, j):
    s = jnp.dot(xn_i, xn_j.T, preferred_element_type=jnp.float32)
    s = jnp.abs(s)
    row = i * T + lax.broadcasted_iota(jnp.int32, (T, T), 0)
    col = j * T + lax.broadcasted_iota(jnp.int32, (T, T), 1)
    return jnp.where(row == col, 0.0, s)


def _count_kernel(lo_ref, xn_i_ref, xn_j_ref, hist_ref, hacc_ref, *, shift):
    i = pl.program_id(0)
    j = pl.program_id(1)
    lo = lo_ref[0, 0]
    s = _sim_tile(xn_i_ref[...], xn_j_ref[...], i, j)
    bits = lax.bitcast_convert_type(s, jnp.int32)
    first = jnp.logical_and(i == 0, j == 0)
    for b in range(NBINS):
        t = lo + (b << shift)
        cnt = jnp.sum((bits >= t).astype(jnp.int32), axis=0)
        prev = jnp.where(first, 0, hacc_ref[b, :])
        hacc_ref[b, :] = prev + cnt

    @pl.when(jnp.logical_and(i == G - 1, j == G - 1))
    def _():
        for b in range(NBINS):
            hist_ref[0, b] = jnp.sum(hacc_ref[b, :])


def _agg_kernel(eps_ref, bout_ref, xn_k_ref, xn_i_ref, h_ref, wl_ref, bl_ref,
                wr_ref, wout_ref, hk_ref, out_ref, acc_ref, cnt_ref):
    k = pl.program_id(0)
    i = pl.program_id(1)
    eps = eps_ref[0, 0]
    s = _sim_tile(xn_k_ref[...], xn_i_ref[...], k, i)
    col = i * T + lax.broadcasted_iota(jnp.int32, (T, T), 1)
    maskf = jnp.where(jnp.logical_and(s >= eps, col < N), 1.0, 0.0)
    part = jnp.dot(maskf, h_ref[...], preferred_element_type=jnp.float32)
    pcnt = jnp.sum(maskf, axis=1, keepdims=True)

    @pl.when(i == 0)
    def _():
        acc_ref[...] = part
        cnt_ref[...] = pcnt

    @pl.when(i > 0)
    def _():
        acc_ref[...] += part
        cnt_ref[...] += pcnt

    @pl.when(i == G - 1)
    def _():
        agg = acc_ref[...] / jnp.maximum(cnt_ref[...], 1.0)
        z = (jnp.dot(agg, wl_ref[...], preferred_element_type=jnp.float32)
             + bl_ref[...]
             + jnp.dot(hk_ref[...], wr_ref[...],
                       preferred_element_type=jnp.float32))
        h2 = jnp.maximum(z, 0.0)
        out_ref[...] = (jnp.dot(h2, wout_ref[...],
                                preferred_element_type=jnp.float32)
                        + bout_ref[0, 0])


def _select(c, lo, shift):
    jstar = jnp.sum((c[0] >= K_RANK).astype(jnp.int32)) - 1
    return lo + (jstar << shift)


def kernel(x, W_in, W_l, b_l, W_r, W_out, b_out):
    f32 = jnp.float32
    xp = jnp.zeros((NP, D), f32).at[:N].set(x.astype(f32))

    xn, h = pl.pallas_call(
        _prep_kernel,
        out_shape=(jax.ShapeDtypeStruct((NP, D), f32),
                   jax.ShapeDtypeStruct((NP, D), f32)),
        interpret=_INTERPRET,
    )(xp, W_in.astype(f32))

    lo = jnp.int32(0)
    for shift in SHIFTS:
        c = pl.pallas_call(
            functools.partial(_count_kernel, shift=shift),
            grid=(G, G),
            in_specs=[pl.BlockSpec((1, 1), lambda i, j: (0, 0),
                                   memory_space=pltpu.SMEM),
                      pl.BlockSpec((T, D), lambda i, j: (i, 0)),
                      pl.BlockSpec((T, D), lambda i, j: (j, 0))],
            out_specs=pl.BlockSpec((1, NBINS), lambda i, j: (0, 0),
                                   memory_space=pltpu.SMEM),
            out_shape=jax.ShapeDtypeStruct((1, NBINS), jnp.int32),
            scratch_shapes=[pltpu.VMEM((NBINS, T), jnp.int32)],
            compiler_params=pltpu.CompilerParams(
                dimension_semantics=("arbitrary", "arbitrary")),
            interpret=_INTERPRET,
        )(lo.reshape(1, 1), xn, xn)
        lo = _select(c, lo, shift)

    eps = lax.bitcast_convert_type(lo, f32)

    out = pl.pallas_call(
        _agg_kernel,
        grid=(G, G),
        in_specs=[pl.BlockSpec((1, 1), lambda k, i: (0, 0),
                               memory_space=pltpu.SMEM),
                  pl.BlockSpec((1, 1), lambda k, i: (0, 0),
                               memory_space=pltpu.SMEM),
                  pl.BlockSpec((T, D), lambda k, i: (k, 0)),
                  pl.BlockSpec((T, D), lambda k, i: (i, 0)),
                  pl.BlockSpec((T, H), lambda k, i: (i, 0)),
                  pl.BlockSpec((H, H), lambda k, i: (0, 0)),
                  pl.BlockSpec((1, H), lambda k, i: (0, 0)),
                  pl.BlockSpec((H, H), lambda k, i: (0, 0)),
                  pl.BlockSpec((H, 1), lambda k, i: (0, 0)),
                  pl.BlockSpec((T, H), lambda k, i: (k, 0))],
        out_specs=pl.BlockSpec((T, 1), lambda k, i: (k, 0)),
        out_shape=jax.ShapeDtypeStruct((NP, 1), f32),
        scratch_shapes=[pltpu.VMEM((T, H), f32),
                        pltpu.VMEM((T, 1), f32)],
        compiler_params=pltpu.CompilerParams(
            dimension_semantics=("arbitrary", "arbitrary")),
        interpret=_INTERPRET,
    )(eps.reshape(1, 1), b_out.astype(f32).reshape(1, 1), xn, xn, h,
      W_l.astype(f32), b_l.astype(f32).reshape(1, H), W_r.astype(f32),
      W_out.astype(f32), h)

    return out[:N]


# speedup vs baseline: 130.3361x; 1.7000x over previous
import functools

import jax
import jax.numpy as jnp
from jax import lax
from jax.experimental import pallas as pl
from jax.experimental.pallas import tpu as pltpu

N = 10000
D = 128
H = 128
NP = 10240
T = 1024
G = NP // T
K_RANK = 320000
NBINS = 8
SHIFTS = (27, 24, 21, 18, 15, 12, 9, 6, 3, 0)

_INTERPRET = False


def _prep_kernel(x_ref, w_ref, xn_ref, h_ref):
    x = x_ref[...]
    nrm = jnp.sqrt(jnp.sum(x * x, axis=1, keepdims=True))
    xn_ref[...] = x / jnp.maximum(nrm, 1e-8)
    h_ref[...] = jnp.maximum(
        jnp.dot(x, w_ref[...], preferred_element_type=jnp.float32), 0.0)


def _sim_tile(xn_i, xn_j, i, j):
    s = jnp.dot(xn_i, xn_j.T, preferred_element_type=jnp.float32)
    s = jnp.abs(s)
    row = i * T + lax.broadcasted_iota(jnp.int32, (T, T), 0)
    col = j * T + lax.broadcasted_iota(jnp.int32, (T, T), 1)
    return jnp.where(row == col, 0.0, s)


def _count_kernel(lo_ref, xn_i_ref, xn_j_ref, hist_ref, hacc_ref, *, shift):
    i = pl.program_id(0)
    j = pl.program_id(1)
    first = jnp.logical_and(i == 0, j == 0)

    @pl.when(first)
    def _():
        hacc_ref[...] = jnp.zeros_like(hacc_ref)

    @pl.when(i <= j)
    def _():
        lo = lo_ref[0, 0]
        s = _sim_tile(xn_i_ref[...], xn_j_ref[...], i, j)
        bits = lax.bitcast_convert_type(s, jnp.int32)
        w = jnp.where(i < j, 2, 1)
        for b in range(1, NBINS):
            t = lo + (b << shift)
            cnt = jnp.sum((bits >= t).astype(jnp.int32), axis=0)
            hacc_ref[b, :] += w * cnt

    @pl.when(jnp.logical_and(i == G - 1, j == G - 1))
    def _():
        for b in range(1, NBINS):
            hist_ref[0, b] = jnp.sum(hacc_ref[b, :])


def _agg_kernel(eps_ref, bout_ref, xn_k_ref, xn_i_ref, h_ref, wl_ref, bl_ref,
                wr_ref, wout_ref, hk_ref, out_ref, acc_ref, cnt_ref):
    k = pl.program_id(0)
    i = pl.program_id(1)
    eps = eps_ref[0, 0]
    s = _sim_tile(xn_k_ref[...], xn_i_ref[...], k, i)
    col = i * T + lax.broadcasted_iota(jnp.int32, (T, T), 1)
    maskf = jnp.where(jnp.logical_and(s >= eps, col < N), 1.0, 0.0)
    part = jnp.dot(maskf, h_ref[...], preferred_element_type=jnp.float32)
    pcnt = jnp.sum(maskf, axis=1, keepdims=True)

    @pl.when(i == 0)
    def _():
        acc_ref[...] = part
        cnt_ref[...] = pcnt

    @pl.when(i > 0)
    def _():
        acc_ref[...] += part
        cnt_ref[...] += pcnt

    @pl.when(i == G - 1)
    def _():
        agg = acc_ref[...] / jnp.maximum(cnt_ref[...], 1.0)
        z = (jnp.dot(agg, wl_ref[...], preferred_element_type=jnp.float32)
             + bl_ref[...]
             + jnp.dot(hk_ref[...], wr_ref[...],
                       preferred_element_type=jnp.float32))
        h2 = jnp.maximum(z, 0.0)
        out_ref[...] = (jnp.dot(h2, wout_ref[...],
                                preferred_element_type=jnp.float32)
                        + bout_ref[0, 0])


def _select(c, lo, shift):
    jstar = jnp.sum((c[0, 1:] >= K_RANK).astype(jnp.int32))
    return lo + (jstar << shift)


def kernel(x, W_in, W_l, b_l, W_r, W_out, b_out):
    f32 = jnp.float32
    xp = jnp.zeros((NP, D), f32).at[:N].set(x.astype(f32))

    xn, h = pl.pallas_call(
        _prep_kernel,
        out_shape=(jax.ShapeDtypeStruct((NP, D), f32),
                   jax.ShapeDtypeStruct((NP, D), f32)),
        interpret=_INTERPRET,
    )(xp, W_in.astype(f32))

    lo = jnp.int32(0)
    for shift in SHIFTS:
        c = pl.pallas_call(
            functools.partial(_count_kernel, shift=shift),
            grid=(G, G),
            in_specs=[pl.BlockSpec((1, 1), lambda i, j: (0, 0),
                                   memory_space=pltpu.SMEM),
                      pl.BlockSpec((T, D), lambda i, j: (i, 0)),
                      pl.BlockSpec((T, D), lambda i, j: (j, 0))],
            out_specs=pl.BlockSpec((1, NBINS), lambda i, j: (0, 0),
                                   memory_space=pltpu.SMEM),
            out_shape=jax.ShapeDtypeStruct((1, NBINS), jnp.int32),
            scratch_shapes=[pltpu.VMEM((NBINS, T), jnp.int32)],
            compiler_params=pltpu.CompilerParams(
                dimension_semantics=("arbitrary", "arbitrary")),
            interpret=_INTERPRET,
        )(lo.reshape(1, 1), xn, xn)
        lo = _select(c, lo, shift)

    eps = lax.bitcast_convert_type(lo, f32)

    out = pl.pallas_call(
        _agg_kernel,
        grid=(G, G),
        in_specs=[pl.BlockSpec((1, 1), lambda k, i: (0, 0),
                               memory_space=pltpu.SMEM),
                  pl.BlockSpec((1, 1), lambda k, i: (0, 0),
                               memory_space=pltpu.SMEM),
                  pl.BlockSpec((T, D), lambda k, i: (k, 0)),
                  pl.BlockSpec((T, D), lambda k, i: (i, 0)),
                  pl.BlockSpec((T, H), lambda k, i: (i, 0)),
                  pl.BlockSpec((H, H), lambda k, i: (0, 0)),
                  pl.BlockSpec((1, H), lambda k, i: (0, 0)),
                  pl.BlockSpec((H, H), lambda k, i: (0, 0)),
                  pl.BlockSpec((H, 1), lambda k, i: (0, 0)),
                  pl.BlockSpec((T, H), lambda k, i: (k, 0))],
        out_specs=pl.BlockSpec((T, 1), lambda k, i: (k, 0)),
        out_shape=jax.ShapeDtypeStruct((NP, 1), f32),
        scratch_shapes=[pltpu.VMEM((T, H), f32),
                        pltpu.VMEM((T, 1), f32)],
        compiler_params=pltpu.CompilerParams(
            dimension_semantics=("arbitrary", "arbitrary")),
        interpret=_INTERPRET,
    )(eps.reshape(1, 1), b_out.astype(f32).reshape(1, 1), xn, xn, h,
      W_l.astype(f32), b_l.astype(f32).reshape(1, H), W_r.astype(f32),
      W_out.astype(f32), h)

    return out[:N]


# speedup vs baseline: 131.8840x; 1.0119x over previous
import jax
import jax.numpy as jnp
from jax import lax
from jax.experimental import pallas as pl
from jax.experimental.pallas import tpu as pltpu

N = 10000
D = 128
H = 128
NP = 10240
T = 1024
G = NP // T
K_RANK = 320000
NBINS = 8
NPASS = 10

_INTERPRET = False


def _prep_kernel(x_ref, w_ref, xn_ref, h_ref):
    x = x_ref[...]
    nrm = jnp.sqrt(jnp.sum(x * x, axis=1, keepdims=True))
    xn_ref[...] = x / jnp.maximum(nrm, 1e-8)
    h_ref[...] = jnp.maximum(
        jnp.dot(x, w_ref[...], preferred_element_type=jnp.float32), 0.0)


def _sim_tile(xn_i, xn_j, i, j):
    s = jnp.dot(xn_i, xn_j.T, preferred_element_type=jnp.float32)
    s = jnp.abs(s)
    row = i * T + lax.broadcasted_iota(jnp.int32, (T, T), 0)
    col = j * T + lax.broadcasted_iota(jnp.int32, (T, T), 1)
    return jnp.where(row == col, 0.0, s)


def _radix_kernel(xn_i_ref, xn_j_ref, eps_ref, hacc_ref, lo_ref):
    p = pl.program_id(0)
    i = pl.program_id(1)
    j = pl.program_id(2)
    first = jnp.logical_and(i == 0, j == 0)
    last = jnp.logical_and(i == G - 1, j == G - 1)
    shift = 27 - 3 * p

    def _selected(prev_shift):
        jstar = jnp.int32(0)
        for b in range(1, NBINS):
            cb = jnp.sum(hacc_ref[b, :])
            jstar += (cb >= K_RANK).astype(jnp.int32)
        return lo_ref[0, 0] + lax.shift_left(jstar, prev_shift)

    @pl.when(jnp.logical_and(first, p == 0))
    def _():
        lo_ref[0, 0] = 0

    @pl.when(jnp.logical_and(first, p > 0))
    def _():
        lo_ref[0, 0] = _selected(shift + 3)

    @pl.when(first)
    def _():
        hacc_ref[...] = jnp.zeros_like(hacc_ref)

    @pl.when(i <= j)
    def _():
        lo = lo_ref[0, 0]
        s = _sim_tile(xn_i_ref[...], xn_j_ref[...], i, j)
        bits = lax.bitcast_convert_type(s, jnp.int32)
        w = jnp.where(i < j, 2, 1)
        for b in range(1, NBINS):
            t = lo + lax.shift_left(jnp.int32(b), shift)
            cnt = jnp.sum((bits >= t).astype(jnp.int32), axis=0)
            hacc_ref[b, :] += w * cnt

    @pl.when(jnp.logical_and(last, p == NPASS - 1))
    def _():
        eps_ref[0, 0] = _selected(0)


def _agg_kernel(eps_ref, bout_ref, xn_k_ref, xn_i_ref, h_ref, wl_ref, bl_ref,
                wr_ref, wout_ref, hk_ref, out_ref, acc_ref, cnt_ref):
    k = pl.program_id(0)
    i = pl.program_id(1)
    eps = eps_ref[0, 0]
    s = _sim_tile(xn_k_ref[...], xn_i_ref[...], k, i)
    col = i * T + lax.broadcasted_iota(jnp.int32, (T, T), 1)
    maskf = jnp.where(jnp.logical_and(s >= eps, col < N), 1.0, 0.0)
    part = jnp.dot(maskf, h_ref[...], preferred_element_type=jnp.float32)
    pcnt = jnp.sum(maskf, axis=1, keepdims=True)

    @pl.when(i == 0)
    def _():
        acc_ref[...] = part
        cnt_ref[...] = pcnt

    @pl.when(i > 0)
    def _():
        acc_ref[...] += part
        cnt_ref[...] += pcnt

    @pl.when(i == G - 1)
    def _():
        agg = acc_ref[...] / jnp.maximum(cnt_ref[...], 1.0)
        z = (jnp.dot(agg, wl_ref[...], preferred_element_type=jnp.float32)
             + bl_ref[...]
             + jnp.dot(hk_ref[...], wr_ref[...],
                       preferred_element_type=jnp.float32))
        h2 = jnp.maximum(z, 0.0)
        out_ref[...] = (jnp.dot(h2, wout_ref[...],
                                preferred_element_type=jnp.float32)
                        + bout_ref[0, 0])


def kernel(x, W_in, W_l, b_l, W_r, W_out, b_out):
    f32 = jnp.float32
    xp = jnp.zeros((NP, D), f32).at[:N].set(x.astype(f32))

    xn, h = pl.pallas_call(
        _prep_kernel,
        out_shape=(jax.ShapeDtypeStruct((NP, D), f32),
                   jax.ShapeDtypeStruct((NP, D), f32)),
        interpret=_INTERPRET,
    )(xp, W_in.astype(f32))

    eps_bits = pl.pallas_call(
        _radix_kernel,
        grid=(NPASS, G, G),
        in_specs=[pl.BlockSpec((T, D), lambda p, i, j: (i, 0)),
                  pl.BlockSpec((T, D), lambda p, i, j: (j, 0))],
        out_specs=pl.BlockSpec((1, 1), lambda p, i, j: (0, 0),
                               memory_space=pltpu.SMEM),
        out_shape=jax.ShapeDtypeStruct((1, 1), jnp.int32),
        scratch_shapes=[pltpu.VMEM((NBINS, T), jnp.int32),
                        pltpu.SMEM((1, 1), jnp.int32)],
        compiler_params=pltpu.CompilerParams(
            dimension_semantics=("arbitrary", "arbitrary", "arbitrary")),
        interpret=_INTERPRET,
    )(xn, xn)

    eps = lax.bitcast_convert_type(eps_bits[0, 0], f32)

    out = pl.pallas_call(
        _agg_kernel,
        grid=(G, G),
        in_specs=[pl.BlockSpec((1, 1), lambda k, i: (0, 0),
                               memory_space=pltpu.SMEM),
                  pl.BlockSpec((1, 1), lambda k, i: (0, 0),
                               memory_space=pltpu.SMEM),
                  pl.BlockSpec((T, D), lambda k, i: (k, 0)),
                  pl.BlockSpec((T, D), lambda k, i: (i, 0)),
                  pl.BlockSpec((T, H), lambda k, i: (i, 0)),
                  pl.BlockSpec((H, H), lambda k, i: (0, 0)),
                  pl.BlockSpec((1, H), lambda k, i: (0, 0)),
                  pl.BlockSpec((H, H), lambda k, i: (0, 0)),
                  pl.BlockSpec((H, 1), lambda k, i: (0, 0)),
                  pl.BlockSpec((T, H), lambda k, i: (k, 0))],
        out_specs=pl.BlockSpec((T, 1), lambda k, i: (k, 0)),
        out_shape=jax.ShapeDtypeStruct((NP, 1), f32),
        scratch_shapes=[pltpu.VMEM((T, H), f32),
                        pltpu.VMEM((T, 1), f32)],
        compiler_params=pltpu.CompilerParams(
            dimension_semantics=("arbitrary", "arbitrary")),
        interpret=_INTERPRET,
    )(eps.reshape(1, 1), b_out.astype(f32).reshape(1, 1), xn, xn, h,
      W_l.astype(f32), b_l.astype(f32).reshape(1, H), W_r.astype(f32),
      W_out.astype(f32), h)

    return out[:N]


# speedup vs baseline: 143.7927x; 1.0903x over previous
import jax
import jax.numpy as jnp
from jax import lax
from jax.experimental import pallas as pl
from jax.experimental.pallas import tpu as pltpu

N = 10000
D = 128
H = 128
NP = 10240
T = 1024
G = NP // T
K_RANK = 320000
NBINS = 4
NPASS = 15

_INTERPRET = False


def _prep_kernel(x_ref, w_ref, xn_ref, h_ref):
    x = x_ref[...]
    nrm = jnp.sqrt(jnp.sum(x * x, axis=1, keepdims=True))
    xn_ref[...] = x / jnp.maximum(nrm, 1e-8)
    h_ref[...] = jnp.maximum(
        jnp.dot(x, w_ref[...], preferred_element_type=jnp.float32), 0.0)


def _sim_tile(xn_i, xn_j, i, j):
    s = jnp.dot(xn_i, xn_j.T, preferred_element_type=jnp.float32)
    s = jnp.abs(s)
    row = i * T + lax.broadcasted_iota(jnp.int32, (T, T), 0)
    col = j * T + lax.broadcasted_iota(jnp.int32, (T, T), 1)
    return jnp.where(row == col, 0.0, s)


def _radix_kernel(xn_i_ref, xn_j_ref, eps_ref, hacc_ref, lo_ref):
    p = pl.program_id(0)
    i = pl.program_id(1)
    j = pl.program_id(2)
    first = jnp.logical_and(i == 0, j == 0)
    last = jnp.logical_and(i == G - 1, j == G - 1)
    shift = 28 - 2 * p

    def _selected(prev_shift):
        jstar = jnp.int32(0)
        for b in range(1, NBINS):
            cb = jnp.sum(hacc_ref[b, :])
            jstar += (cb >= K_RANK).astype(jnp.int32)
        return lo_ref[0, 0] + lax.shift_left(jstar, prev_shift)

    @pl.when(jnp.logical_and(first, p == 0))
    def _():
        lo_ref[0, 0] = 0

    @pl.when(jnp.logical_and(first, p > 0))
    def _():
        lo_ref[0, 0] = _selected(shift + 2)

    @pl.when(first)
    def _():
        hacc_ref[...] = jnp.zeros_like(hacc_ref)

    @pl.when(i <= j)
    def _():
        lo = lo_ref[0, 0]
        s = _sim_tile(xn_i_ref[...], xn_j_ref[...], i, j)
        bits = lax.bitcast_convert_type(s, jnp.int32)
        w = jnp.where(i < j, 2, 1)
        for b in range(1, NBINS):
            t = lo + lax.shift_left(jnp.int32(b), shift)
            cnt = jnp.sum((bits >= t).astype(jnp.int32), axis=0)
            hacc_ref[b, :] += w * cnt

    @pl.when(jnp.logical_and(last, p == NPASS - 1))
    def _():
        eps_ref[0, 0] = _selected(0)


def _agg_kernel(eps_ref, bout_ref, xn_k_ref, xn_i_ref, h_ref, wl_ref, bl_ref,
                wr_ref, wout_ref, hk_ref, out_ref, acc_ref, cnt_ref):
    k = pl.program_id(0)
    i = pl.program_id(1)
    eps = eps_ref[0, 0]
    s = _sim_tile(xn_k_ref[...], xn_i_ref[...], k, i)
    col = i * T + lax.broadcasted_iota(jnp.int32, (T, T), 1)
    maskf = jnp.where(jnp.logical_and(s >= eps, col < N), 1.0, 0.0)
    part = jnp.dot(maskf, h_ref[...], preferred_element_type=jnp.float32)
    pcnt = jnp.sum(maskf, axis=1, keepdims=True)

    @pl.when(i == 0)
    def _():
        acc_ref[...] = part
        cnt_ref[...] = pcnt

    @pl.when(i > 0)
    def _():
        acc_ref[...] += part
        cnt_ref[...] += pcnt

    @pl.when(i == G - 1)
    def _():
        agg = acc_ref[...] / jnp.maximum(cnt_ref[...], 1.0)
        z = (jnp.dot(agg, wl_ref[...], preferred_element_type=jnp.float32)
             + bl_ref[...]
             + jnp.dot(hk_ref[...], wr_ref[...],
                       preferred_element_type=jnp.float32))
        h2 = jnp.maximum(z, 0.0)
        out_ref[...] = (jnp.dot(h2, wout_ref[...],
                                preferred_element_type=jnp.float32)
                        + bout_ref[0, 0])


def kernel(x, W_in, W_l, b_l, W_r, W_out, b_out):
    f32 = jnp.float32
    xp = jnp.zeros((NP, D), f32).at[:N].set(x.astype(f32))

    xn, h = pl.pallas_call(
        _prep_kernel,
        out_shape=(jax.ShapeDtypeStruct((NP, D), f32),
                   jax.ShapeDtypeStruct((NP, D), f32)),
        interpret=_INTERPRET,
    )(xp, W_in.astype(f32))

    eps_bits = pl.pallas_call(
        _radix_kernel,
        grid=(NPASS, G, G),
        in_specs=[pl.BlockSpec((T, D), lambda p, i, j: (i, 0)),
                  pl.BlockSpec((T, D), lambda p, i, j: (j, 0))],
        out_specs=pl.BlockSpec((1, 1), lambda p, i, j: (0, 0),
                               memory_space=pltpu.SMEM),
        out_shape=jax.ShapeDtypeStruct((1, 1), jnp.int32),
        scratch_shapes=[pltpu.VMEM((NBINS, T), jnp.int32),
                        pltpu.SMEM((1, 1), jnp.int32)],
        compiler_params=pltpu.CompilerParams(
            dimension_semantics=("arbitrary", "arbitrary", "arbitrary")),
        interpret=_INTERPRET,
    )(xn, xn)

    eps = lax.bitcast_convert_type(eps_bits[0, 0], f32)

    out = pl.pallas_call(
        _agg_kernel,
        grid=(G, G),
        in_specs=[pl.BlockSpec((1, 1), lambda k, i: (0, 0),
                               memory_space=pltpu.SMEM),
                  pl.BlockSpec((1, 1), lambda k, i: (0, 0),
                               memory_space=pltpu.SMEM),
                  pl.BlockSpec((T, D), lambda k, i: (k, 0)),
                  pl.BlockSpec((T, D), lambda k, i: (i, 0)),
                  pl.BlockSpec((T, H), lambda k, i: (i, 0)),
                  pl.BlockSpec((H, H), lambda k, i: (0, 0)),
                  pl.BlockSpec((1, H), lambda k, i: (0, 0)),
                  pl.BlockSpec((H, H), lambda k, i: (0, 0)),
                  pl.BlockSpec((H, 1), lambda k, i: (0, 0)),
                  pl.BlockSpec((T, H), lambda k, i: (k, 0))],
        out_specs=pl.BlockSpec((T, 1), lambda k, i: (k, 0)),
        out_shape=jax.ShapeDtypeStruct((NP, 1), f32),
        scratch_shapes=[pltpu.VMEM((T, H), f32),
                        pltpu.VMEM((T, 1), f32)],
        compiler_params=pltpu.CompilerParams(
            dimension_semantics=("arbitrary", "arbitrary")),
        interpret=_INTERPRET,
    )(eps.reshape(1, 1), b_out.astype(f32).reshape(1, 1), xn, xn, h,
      W_l.astype(f32), b_l.astype(f32).reshape(1, H), W_r.astype(f32),
      W_out.astype(f32), h)

    return out[:N]
